# baseline (speedup 1.0000x reference)
import functools

import jax
import jax.numpy as jnp
from jax import lax
from jax.experimental import pallas as pl
from jax.experimental.pallas import tpu as pltpu
from jax.experimental.pallas import tpu_sc as plsc

D = H = W = 256
N_RAYS, N_BINS = 16384, 128
N = N_RAYS * N_BINS
NC, NS, L = 2, 16, 16
NW = NC * NS
P = N // NW
C = 4096
NCHUNK = P // C


def _floor_i32(x):
    t = x.astype(jnp.int32)
    tf = t.astype(jnp.float32)
    return jnp.where(tf > x, t - 1, t)


def _axis(c, extent):
    x = c * (0.5 * extent) + (0.5 * extent - 0.5)
    i0 = _floor_i32(x)
    i1 = i0 + 1
    w1 = x - i0.astype(jnp.float32)
    w0 = 1.0 - w1
    hi = extent - 1
    v0 = (i0 >= 0) & (i0 <= hi)
    v1 = (i1 >= 0) & (i1 <= hi)
    w0 = jnp.where(v0, w0, 0.0)
    w1 = jnp.where(v1, w1, 0.0)
    c0 = jnp.minimum(jnp.maximum(i0, 0), hi)
    c1 = jnp.minimum(jnp.maximum(i1, 0), hi)
    return c0, c1, w0, w1


def _body(grid_hbm, coords_hbm, out_hbm, coords_v, idx_v, w_v, vals_v,
          out_v, sem):
    wid = lax.axis_index("s") * NC + lax.axis_index("c")
    base_pt = wid * P
    lanes = lax.iota(jnp.int32, L)

    def chunk_body(ci, carry):
        pt0 = base_pt + ci * C
        pltpu.sync_copy(coords_hbm.at[pl.ds(pt0 * 3, 3 * C)], coords_v)

        def grp(g, carry2):
            o = g * L
            p3 = (o + lanes) * 3
            cx = plsc.load_gather(coords_v, [p3])
            cy = plsc.load_gather(coords_v, [p3 + 1])
            cz = plsc.load_gather(coords_v, [p3 + 2])
            x0, x1, wx0, wx1 = _axis(cx, W)
            y0, y1, wy0, wy1 = _axis(cy, H)
            z0, z1, wz0, wz1 = _axis(cz, D)
            iz0 = z0 * (H * W)
            iz1 = z1 * (H * W)
            iy0 = y0 * W
            iy1 = y1 * W
            w00 = wz0 * wy0
            w01 = wz0 * wy1
            w10 = wz1 * wy0
            w11 = wz1 * wy1
            corners = (
                (iz0 + iy0 + x0, w00 * wx0),
                (iz0 + iy0 + x1, w00 * wx1),
                (iz0 + iy1 + x0, w01 * wx0),
                (iz0 + iy1 + x1, w01 * wx1),
                (iz1 + iy0 + x0, w10 * wx0),
                (iz1 + iy0 + x1, w10 * wx1),
                (iz1 + iy1 + x0, w11 * wx0),
                (iz1 + iy1 + x1, w11 * wx1),
            )
            for k, (iv, wv) in enumerate(corners):
                idx_v[pl.ds(k * C + o, L)] = iv
                w_v[pl.ds(k * C + o, L)] = wv
            return carry2

        lax.fori_loop(0, C // L, grp, 0, unroll=2)

        pltpu.async_copy(grid_hbm.at[idx_v], vals_v, sem).wait()

        def grp2(g, carry2):
            o = g * L
            acc = vals_v[pl.ds(o, L)] * w_v[pl.ds(o, L)]
            for k in range(1, 8):
                acc = acc + vals_v[pl.ds(k * C + o, L)] * w_v[pl.ds(k * C + o, L)]
            out_v[pl.ds(o, L)] = acc
            return carry2

        lax.fori_loop(0, C // L, grp2, 0, unroll=2)
        pltpu.sync_copy(out_v, out_hbm.at[pl.ds(pt0, C)])
        return carry

    lax.fori_loop(0, NCHUNK, chunk_body, 0)


@jax.jit
def kernel(occupancy_grid, ray_bin_centers):
    grid_flat = occupancy_grid.reshape(D * H * W)
    coords_flat = ray_bin_centers.reshape(N * 3)
    mesh = plsc.VectorSubcoreMesh(core_axis_name="c", subcore_axis_name="s")
    run = pl.kernel(
        _body,
        out_type=jax.ShapeDtypeStruct((N,), jnp.float32),
        mesh=mesh,
        compiler_params=pltpu.CompilerParams(needs_layout_passes=False),
        scratch_types=[
            pltpu.VMEM((3 * C,), jnp.float32),
            pltpu.VMEM((8 * C,), jnp.int32),
            pltpu.VMEM((8 * C,), jnp.float32),
            pltpu.VMEM((8 * C,), jnp.float32),
            pltpu.VMEM((C,), jnp.float32),
            pltpu.SemaphoreType.DMA,
        ],
    )
    out = run(grid_flat, coords_flat)
    return out.reshape(N_RAYS, N_BINS)
